# Initial kernel scaffold; baseline (speedup 1.0000x reference)
import jax
import jax.numpy as jnp
from jax.experimental import pallas as pl


def kernel(x, row_embed, col_embed):
    raise NotImplementedError("write your pallas kernel here")


# speedup vs baseline: 3.7950x; 3.7950x over previous
import jax
import jax.numpy as jnp
from jax.experimental import pallas as pl
from jax.experimental.pallas import tpu as pltpu

_CB = 32


def _body(nrow_blocks, x_ref, emb_ref, o_ref):
    ci = pl.program_id(1)

    @pl.when(ci < nrow_blocks)
    def _():
        o_ref[...] = x_ref[...] + emb_ref[...][None, :, :, None]

    @pl.when(ci >= nrow_blocks)
    def _():
        o_ref[...] = x_ref[...] + emb_ref[...][None, :, None, :]


def kernel(x, row_embed, col_embed):
    b, c, h, w = x.shape
    chalf = c // 2
    assert chalf % _CB == 0
    nrow_blocks = chalf // _CB
    emb = jnp.concatenate([row_embed.T, col_embed.T], axis=0)

    import functools
    grid = (b, c // _CB)
    return pl.pallas_call(
        functools.partial(_body, nrow_blocks),
        grid=grid,
        in_specs=[
            pl.BlockSpec((1, _CB, h, w), lambda bi, ci: (bi, ci, 0, 0)),
            pl.BlockSpec((_CB, emb.shape[1]), lambda bi, ci: (ci, 0)),
        ],
        out_specs=pl.BlockSpec((1, _CB, h, w), lambda bi, ci: (bi, ci, 0, 0)),
        out_shape=jax.ShapeDtypeStruct(x.shape, x.dtype),
        compiler_params=pltpu.CompilerParams(
            dimension_semantics=("parallel", "arbitrary"),
        ),
    )(x, emb)


# speedup vs baseline: 3.8135x; 1.0049x over previous
import jax
import jax.numpy as jnp
from jax.experimental import pallas as pl
from jax.experimental.pallas import tpu as pltpu

_CB = 48


def _body(nrow_blocks, x_ref, emb_ref, o_ref):
    ci = pl.program_id(1)

    @pl.when(ci < nrow_blocks)
    def _():
        o_ref[...] = x_ref[...] + emb_ref[...][None, :, :, None]

    @pl.when(ci >= nrow_blocks)
    def _():
        o_ref[...] = x_ref[...] + emb_ref[...][None, :, None, :]


def kernel(x, row_embed, col_embed):
    b, c, h, w = x.shape
    chalf = c // 2
    assert chalf % _CB == 0
    nrow_blocks = chalf // _CB
    emb = jnp.concatenate([row_embed.T, col_embed.T], axis=0)

    import functools
    grid = (b, c // _CB)
    return pl.pallas_call(
        functools.partial(_body, nrow_blocks),
        grid=grid,
        in_specs=[
            pl.BlockSpec((1, _CB, h, w), lambda bi, ci: (bi, ci, 0, 0)),
            pl.BlockSpec((_CB, emb.shape[1]), lambda bi, ci: (ci, 0)),
        ],
        out_specs=pl.BlockSpec((1, _CB, h, w), lambda bi, ci: (bi, ci, 0, 0)),
        out_shape=jax.ShapeDtypeStruct(x.shape, x.dtype),
        compiler_params=pltpu.CompilerParams(
            dimension_semantics=("parallel", "arbitrary"),
        ),
    )(x, emb)
